# Initial kernel scaffold; baseline (speedup 1.0000x reference)
import functools

import jax
import jax.numpy as jnp
from jax import lax
from jax.experimental import pallas as pl
from jax.experimental.pallas import tpu as pltpu
from jax.experimental.pallas import tpu_sc as plsc

B = 16384
F = 26
V = 1000
D = 50
N = B * F

NC = 2
NS = 16
NW = NC * NS
NPW = N // NW
SC_CHUNK = 208
STREAM = 104
NG = NPW // SC_CHUNK


def _body(idx_hbm, tab_hbm, out_hbm, idx_v, offs_v, rows_v, sem):
    wid = lax.axis_index("s") * NC + lax.axis_index("c")
    base = wid * NPW

    pltpu.sync_copy(idx_hbm.at[pl.ds(base, NPW)], idx_v)

    for j in range(SC_CHUNK // 16):
        pos = lax.iota(jnp.int32, 16) + (j * 16)
        offs_v[pl.ds(j * 16, 16)] = (pos % F) * V

    def g_body(g, carry):
        s0 = pl.multiple_of(g * SC_CHUNK, SC_CHUNK)
        for j in range(SC_CHUNK // 16):
            o = pl.multiple_of(s0 + j * 16, 16)
            idx_v[pl.ds(o, 16)] = idx_v[pl.ds(o, 16)] + offs_v[pl.ds(j * 16, 16)]
        cp0 = pltpu.make_async_copy(
            tab_hbm.at[idx_v.at[pl.ds(s0, STREAM)]],
            rows_v.at[pl.ds(0, STREAM)], sem)
        cp1 = pltpu.make_async_copy(
            tab_hbm.at[idx_v.at[pl.ds(s0 + STREAM, STREAM)]],
            rows_v.at[pl.ds(STREAM, STREAM)], sem)
        cp0.start()
        cp1.start()
        cp0.wait()
        cp1.wait()
        pltpu.sync_copy(rows_v, out_hbm.at[pl.ds(base + s0, SC_CHUNK)])
        return carry

    lax.fori_loop(0, NG, g_body, 0)


@functools.partial(
    pl.kernel,
    out_type=jax.ShapeDtypeStruct((N, D), jnp.float32),
    mesh=plsc.VectorSubcoreMesh(core_axis_name="c", subcore_axis_name="s"),
    scratch_types=[
        pltpu.VMEM((NPW,), jnp.int32),
        pltpu.VMEM((SC_CHUNK,), jnp.int32),
        pltpu.VMEM((SC_CHUNK, D), jnp.float32),
        pltpu.SemaphoreType.DMA,
    ],
)
def _gather_kernel(idx_hbm, tab_hbm, out_hbm, idx_v, offs_v, rows_v, sem):
    _body(idx_hbm, tab_hbm, out_hbm, idx_v, offs_v, rows_v, sem)


def kernel(indices, tables):
    idx_flat = indices.astype(jnp.int32).reshape(N)
    tab_flat = tables.reshape(F * V, D)
    out = _gather_kernel(idx_flat, tab_flat)
    return out.reshape(B, F * D)


# speedup vs baseline: 3.0340x; 3.0340x over previous
import functools

import jax
import jax.numpy as jnp
from jax import lax
from jax.experimental import pallas as pl
from jax.experimental.pallas import tpu as pltpu
from jax.experimental.pallas import tpu_sc as plsc

B = 16384
F = 26
V = 1000
D = 50
DP = 56
N = B * F

NC = 2
NS = 16
NW = NC * NS
NPW = N // NW
STREAM = 104
SC_CHUNK = 2 * STREAM
NG = NPW // SC_CHUNK


def _body(idx_hbm, tab_hbm, out_hbm, idx_v, rows_v, sem):
    wid = lax.axis_index("s") * NC + lax.axis_index("c")
    base = wid * NPW

    pltpu.sync_copy(idx_hbm.at[pl.ds(base, NPW)], idx_v)

    def g_body(g, carry):
        s0 = pl.multiple_of(g * SC_CHUNK, SC_CHUNK)
        cp0 = pltpu.make_async_copy(
            tab_hbm.at[idx_v.at[pl.ds(s0, STREAM)]],
            rows_v.at[pl.ds(0, STREAM)], sem)
        cp1 = pltpu.make_async_copy(
            tab_hbm.at[idx_v.at[pl.ds(s0 + STREAM, STREAM)]],
            rows_v.at[pl.ds(STREAM, STREAM)], sem)
        cp0.start()
        cp1.start()
        cp0.wait()
        cp1.wait()
        pltpu.sync_copy(rows_v, out_hbm.at[pl.ds(base + s0, SC_CHUNK)])
        return carry

    lax.fori_loop(0, NG, g_body, 0)


@functools.partial(
    pl.kernel,
    out_type=jax.ShapeDtypeStruct((N, DP), jnp.float32),
    mesh=plsc.VectorSubcoreMesh(core_axis_name="c", subcore_axis_name="s"),
    compiler_params=pltpu.CompilerParams(use_tc_tiling_on_sc=False),
    scratch_types=[
        pltpu.VMEM((NPW,), jnp.int32),
        pltpu.VMEM((SC_CHUNK, DP), jnp.float32),
        pltpu.SemaphoreType.DMA,
    ],
)
def _gather_kernel(idx_hbm, tab_hbm, out_hbm, idx_v, rows_v, sem):
    _body(idx_hbm, tab_hbm, out_hbm, idx_v, rows_v, sem)


def kernel(indices, tables):
    idx_flat = (indices.astype(jnp.int32)
                + jnp.arange(F, dtype=jnp.int32)[None, :] * V).reshape(N)
    tab_pad = jnp.pad(tables.reshape(F * V, D), ((0, 0), (0, DP - D)))
    out = _gather_kernel(idx_flat, tab_pad)
    return out[:, :D].reshape(B, F * D)
